# baseline (speedup 1.0000x reference)
import functools
import math

import jax
import jax.numpy as jnp
from jax import lax
from jax.experimental import pallas as pl
from jax.experimental.pallas import tpu as pltpu
from jax.experimental.pallas import tpu_sc as plsc

D_MODEL = 64
SCALE = math.sqrt(D_MODEL)
ROWS_PER_CHUNK = 2
NBUF = 4


@functools.lru_cache(maxsize=None)
def _build(BATCH: int, SEQ: int, V: int):
    info = plsc.get_sparse_core_info()
    NC, NS, L = info.num_cores, info.num_subcores, info.num_lanes
    NW = NC * NS
    C = ROWS_PER_CHUNK * SEQ
    assert BATCH % (NW * ROWS_PER_CHUNK) == 0
    R = BATCH // (NW * ROWS_PER_CHUNK)
    assert R % NBUF == 0 and R > NBUF
    G = R // NBUF
    NIDX = BATCH // ROWS_PER_CHUNK
    mesh = plsc.VectorSubcoreMesh(core_axis_name="c", subcore_axis_name="s")

    @functools.partial(
        pl.kernel,
        mesh=mesh,
        out_type=jax.ShapeDtypeStruct((BATCH, SEQ, D_MODEL), jnp.float32),
        compiler_params=pltpu.CompilerParams(use_tc_tiling_on_sc=False),
        scratch_types=[
            pltpu.VMEM((R, C), jnp.int32),
            pltpu.VMEM((NBUF, C, D_MODEL), jnp.float32),
            pltpu.SemaphoreType.DMA,
            pltpu.SemaphoreType.DMA,
        ],
    )
    def k(table_hbm, idx_hbm, out_hbm, idx_v, rows_v, gsem, osem):
        wid = lax.axis_index("s") * NC + lax.axis_index("c")
        r0 = wid * R
        pltpu.sync_copy(idx_hbm.at[pl.ds(r0, R)], idx_v)

        def gather(j, b):
            pltpu.async_copy(table_hbm.at[idx_v.at[j]], rows_v.at[b], gsem)

        def wait_gather(j, b):
            pltpu.make_async_copy(
                table_hbm.at[idx_v.at[j]], rows_v.at[b], gsem
            ).wait()

        def out_copies(j, b):
            q0 = (r0 + j) * ROWS_PER_CHUNK
            for p in range(ROWS_PER_CHUNK):
                pltpu.async_copy(
                    rows_v.at[b, pl.ds(p * SEQ, SEQ)],
                    out_hbm.at[q0 + p],
                    osem,
                )

        def drain_one_out(b):
            for p in range(ROWS_PER_CHUNK):
                pltpu.make_async_copy(
                    rows_v.at[b, pl.ds(p * SEQ, SEQ)],
                    out_hbm.at[0],
                    osem,
                ).wait()

        def scale(b):
            def row_body(r, _):
                for c in range(D_MODEL // L):
                    rows_v[b, r, pl.ds(c * L, L)] = (
                        rows_v[b, r, pl.ds(c * L, L)] * SCALE
                    )
                return ()

            lax.fori_loop(0, C, row_body, ())

        for b in range(NBUF):
            gather(b, b)

        def group_body(g, _):
            for b in range(NBUF):
                j = g * NBUF + b
                wait_gather(j, b)
                scale(b)
                out_copies(j, b)
                bp = (b - 1) % NBUF
                cond = (g >= 1) if b == 0 else (g < G - 1)

                @pl.when(cond)
                def _():
                    drain_one_out(bp)
                    gather(j - 1 + NBUF, bp)

            return ()

        lax.fori_loop(0, G, group_body, ())

        for b in range(NBUF):
            drain_one_out(b)

    return k


def kernel(x, table):
    BATCH, SEQ = x.shape
    idx = x.reshape(BATCH // ROWS_PER_CHUNK, ROWS_PER_CHUNK * SEQ).astype(
        jnp.int32
    )
    return _build(BATCH, SEQ, table.shape[0])(table, idx)


# speedup vs baseline: 1.0188x; 1.0188x over previous
import functools
import math

import jax
import jax.numpy as jnp
from jax import lax
from jax.experimental import pallas as pl
from jax.experimental.pallas import tpu as pltpu
from jax.experimental.pallas import tpu_sc as plsc

D_MODEL = 64
SCALE = math.sqrt(D_MODEL)
CHUNK = 128
NBUF = 4


@functools.lru_cache(maxsize=None)
def _build(B: int, V: int):
    info = plsc.get_sparse_core_info()
    NC, NS, L = info.num_cores, info.num_subcores, info.num_lanes
    NW = NC * NS
    assert B % (NW * CHUNK) == 0
    R = B // (NW * CHUNK)
    assert R % NBUF == 0 and R > NBUF
    G = R // NBUF
    H = CHUNK // 2
    mesh = plsc.VectorSubcoreMesh(core_axis_name="c", subcore_axis_name="s")

    @functools.partial(
        pl.kernel,
        mesh=mesh,
        out_type=jax.ShapeDtypeStruct((B // 2, 2 * D_MODEL), jnp.float32),
        compiler_params=pltpu.CompilerParams(use_tc_tiling_on_sc=False),
        scratch_types=[
            pltpu.VMEM((R, CHUNK), jnp.int32),
            pltpu.VMEM((NBUF, CHUNK, D_MODEL), jnp.float32),
            pltpu.VMEM((NBUF, H, 2 * D_MODEL), jnp.float32),
            pltpu.SemaphoreType.DMA,
            pltpu.SemaphoreType.DMA,
        ],
    )
    def k(table_hbm, idx_hbm, out_hbm, idx_v, rows_v, obuf_v, gsem, osem):
        wid = lax.axis_index("s") * NC + lax.axis_index("c")
        r0 = wid * R
        pltpu.sync_copy(idx_hbm.at[pl.ds(r0, R)], idx_v)

        def gather(j, b):
            pltpu.async_copy(table_hbm.at[idx_v.at[j]], rows_v.at[b], gsem)

        def wait_gather(j, b):
            pltpu.make_async_copy(
                table_hbm.at[idx_v.at[j]], rows_v.at[b], gsem
            ).wait()

        def drain_one_out(b):
            pltpu.make_async_copy(
                obuf_v.at[b], out_hbm.at[pl.ds(0, H)], osem
            ).wait()

        def scale_repack(b):
            def pair_body(t, _):
                for h in range(2):
                    for c in range(D_MODEL // L):
                        obuf_v[b, t, pl.ds(h * D_MODEL + c * L, L)] = (
                            rows_v[b, 2 * t + h, pl.ds(c * L, L)] * SCALE
                        )
                return ()

            lax.fori_loop(0, H, pair_body, ())

        for b in range(NBUF):
            gather(b, b)

        def group_body(g, _):
            for b in range(NBUF):
                j = g * NBUF + b
                wait_gather(j, b)
                @pl.when(g >= 1)
                def _():
                    drain_one_out(b)

                scale_repack(b)
                @pl.when(g < G - 1)
                def _():
                    gather(j + NBUF, b)

                pltpu.async_copy(
                    obuf_v.at[b], out_hbm.at[pl.ds((r0 + j) * H, H)], osem
                )
            return ()

        lax.fori_loop(0, G, group_body, ())

        for b in range(NBUF):
            drain_one_out(b)

    return k


def kernel(x, table):
    B = x.shape[0] * x.shape[1]
    idx = x.reshape(B // CHUNK, CHUNK).astype(jnp.int32)
    out2 = _build(B, table.shape[0])(table, idx)
    return out2.reshape(x.shape + (D_MODEL,))
